# Initial kernel scaffold; baseline (speedup 1.0000x reference)
import jax
import jax.numpy as jnp
from jax.experimental import pallas as pl


def kernel(x, edge_index, W, b, W2, b2):
    raise NotImplementedError("write your pallas kernel here")


# speedup vs baseline: 105.2731x; 105.2731x over previous
import functools

import jax
import jax.numpy as jnp
from jax import lax
from jax.experimental import pallas as pl
from jax.experimental.pallas import tpu as pltpu, tpu_sc as plsc

N_NODES = 10000
N_EDGES = 320000
IN_DIM = 128
NPAD = 10240
NROW = NPAD // 128

_MESH = plsc.VectorSubcoreMesh(core_axis_name="c", subcore_axis_name="s")
_NW = _MESH.num_cores * _MESH.num_subcores
_EPW = N_EDGES // _NW
_SC_PARAMS = pltpu.CompilerParams(needs_layout_passes=False)


def _zero_f32(ref, n):
    z = jnp.zeros((16,), jnp.float32)

    def body(i, _):
        ref[pl.ds(i * 16, 16)] = z
        return 0

    lax.fori_loop(0, n // 16, body, 0)


@functools.partial(
    pl.kernel,
    out_type=jax.ShapeDtypeStruct((_NW, NPAD), jnp.float32),
    mesh=_MESH,
    scratch_types=[
        pltpu.VMEM((_EPW,), jnp.int32),
        pltpu.VMEM((NPAD,), jnp.float32),
    ],
    compiler_params=_SC_PARAMS,
)
def _sc_degree(dst_hbm, degp_hbm, dst_v, deg_v):
    wid = lax.axis_index("s") * _MESH.num_cores + lax.axis_index("c")
    pltpu.sync_copy(dst_hbm.at[pl.ds(wid * _EPW, _EPW)], dst_v)
    _zero_f32(deg_v, NPAD)
    ones = jnp.ones((16,), jnp.float32)

    def body(i, _):
        d = dst_v[pl.ds(i * 16, 16)]
        plsc.addupdate_scatter(deg_v, [d], ones)
        return 0

    lax.fori_loop(0, _EPW // 16, body, 0)
    pltpu.sync_copy(deg_v, degp_hbm.at[wid])


@functools.partial(
    pl.kernel,
    out_type=jax.ShapeDtypeStruct((_NW, NPAD), jnp.float32),
    mesh=_MESH,
    scratch_types=[
        pltpu.VMEM((_EPW,), jnp.int32),
        pltpu.VMEM((_EPW,), jnp.int32),
        pltpu.VMEM((NPAD,), jnp.float32),
        pltpu.VMEM((NPAD,), jnp.float32),
    ],
    compiler_params=_SC_PARAMS,
)
def _sc_accumulate(src_hbm, dst_hbm, zdis_hbm, accp_hbm, src_v, dst_v, zdis_v, acc_v):
    wid = lax.axis_index("s") * _MESH.num_cores + lax.axis_index("c")
    pltpu.sync_copy(src_hbm.at[pl.ds(wid * _EPW, _EPW)], src_v)
    pltpu.sync_copy(dst_hbm.at[pl.ds(wid * _EPW, _EPW)], dst_v)
    pltpu.sync_copy(zdis_hbm, zdis_v)
    _zero_f32(acc_v, NPAD)

    def body(i, _):
        s = src_v[pl.ds(i * 16, 16)]
        vals = plsc.load_gather(zdis_v, [s])
        d = dst_v[pl.ds(i * 16, 16)]
        plsc.addupdate_scatter(acc_v, [d], vals)
        return 0

    lax.fori_loop(0, _EPW // 16, body, 0)
    pltpu.sync_copy(acc_v, accp_hbm.at[wid])


def _tc_mid_body(x3_ref, w_ref, w2_ref, b_ref, b2_ref, degp_ref,
                 zdis_ref, dis_ref, sterm_ref):
    wc = w_ref[...] @ w2_ref[...]
    z = jax.lax.dot_general(
        x3_ref[...], wc[:, 0],
        dimension_numbers=(((2,), (0,)), ((), ())),
    )
    deg = 1.0 + jnp.sum(degp_ref[...], axis=0)
    dis = jax.lax.rsqrt(deg)
    c = jnp.sum(b_ref[...] * w2_ref[...][:, 0]) + b2_ref[0]
    zdis_ref[...] = z * dis
    dis_ref[...] = dis
    sterm_ref[...] = z / deg + c


def _tc_final_body(accp_ref, dis_ref, sterm_ref, out_ref):
    acc = jnp.sum(accp_ref[...], axis=0)
    out_ref[...] = jax.nn.sigmoid(dis_ref[...] * acc + sterm_ref[...])


_tc_mid = pl.pallas_call(
    _tc_mid_body,
    out_shape=[
        jax.ShapeDtypeStruct((NROW, 128), jnp.float32),
        jax.ShapeDtypeStruct((NROW, 128), jnp.float32),
        jax.ShapeDtypeStruct((NROW, 128), jnp.float32),
    ],
)

_tc_final = pl.pallas_call(
    _tc_final_body,
    out_shape=jax.ShapeDtypeStruct((NROW, 128), jnp.float32),
)


def kernel(x, edge_index, W, b, W2, b2):
    src = edge_index[0].astype(jnp.int32)
    dst = edge_index[1].astype(jnp.int32)
    x3 = jnp.pad(x, ((0, NPAD - N_NODES), (0, 0))).reshape(NROW, 128, IN_DIM)

    degp = _sc_degree(dst)
    zdis, dis, sterm = _tc_mid(
        x3, W, W2, b, b2, degp.reshape(_NW, NROW, 128)
    )
    accp = _sc_accumulate(src, dst, zdis.reshape(NPAD))
    score = _tc_final(accp.reshape(_NW, NROW, 128), dis, sterm)
    return score.reshape(NPAD)[:N_NODES]
